# baseline (speedup 1.0000x reference)
import functools

import jax
import jax.numpy as jnp
from jax.experimental import pallas as pl
from jax.experimental.pallas import tpu as pltpu


def _body(x_ref, y_ref, o_ref, *, C):
    x = x_ref[...]
    yv = y_ref[...]
    R = x.shape[0]
    col = jax.lax.broadcasted_iota(jnp.int32, (R, C), 1)
    onehot = col == yv[:, None]
    oy = jnp.sum(jnp.where(onehot, x, 0.0), axis=1, keepdims=True)
    hinge = jnp.maximum(x - oy + 1.0, 0.0)
    o_ref[...] = (jnp.sum(hinge, axis=1) - 1.0) * (1.0 / C)


def kernel(output, y):
    B, C = output.shape
    R = 4096
    grid = (B // R,)
    return pl.pallas_call(
        functools.partial(_body, C=C),
        grid=grid,
        in_specs=[
            pl.BlockSpec((R, C), lambda i: (i, 0)),
            pl.BlockSpec((R,), lambda i: (i,)),
        ],
        out_specs=pl.BlockSpec((R,), lambda i: (i,)),
        out_shape=jax.ShapeDtypeStruct((B,), jnp.float32),
    )(output, y)
